# baseline (speedup 1.0000x reference)
import functools

import jax
import jax.numpy as jnp
from jax import lax
from jax.experimental import pallas as pl
from jax.experimental.pallas import tpu as pltpu
from jax.experimental.pallas import tpu_sc as plsc

F32 = jnp.float32

N = 10000
E = 320000
EMB = 16
NC, NS = 2, 16
NW = NC * NS
EPT = 10240
PADE = NW * EPT
CH = 128
SS = 512
NROW = 10112
STRIPE = NROW // NS


def _prep_body(x8, wm2, ws2, bs2, u, w1, b1, xm8_o, xsb8_o, us_o):
    x8v = x8[...]
    wm = wm2[...]
    ws = ws2[...]
    bs = bs2[...]
    for j in range(8):
        xj = x8v[:, j * 128:(j + 1) * 128]
        xm8_o[:, j * EMB:(j + 1) * EMB] = jnp.dot(
            xj, wm, preferred_element_type=F32)
        xsb8_o[:, j * EMB:(j + 1) * EMB] = jnp.dot(
            xj, ws, preferred_element_type=F32) + bs
    us_o[...] = jax.nn.relu(
        jnp.dot(u[...], w1[...], preferred_element_type=F32) + b1[...])


def _edge_body(xm_hbm, src3_hbm, dst3_hbm, ea_hbm, zer_hbm,
               agg_hbm, t2_hbm,
               src2d, dst2d, ea2d0, ea2d1, earows0, earows1,
               xmbuf0, xmbuf1, stripe, aggsh, t2sh, sg0, sg1, ss0, ss1):
    cid = lax.axis_index("c")
    sid = lax.axis_index("s")
    wid = sid * NC + cid

    pltpu.sync_copy(zer_hbm, stripe)
    pltpu.sync_copy(stripe, aggsh.at[pl.ds(sid * STRIPE, STRIPE)])
    pltpu.sync_copy(stripe, t2sh.at[pl.ds(sid * STRIPE, STRIPE)])
    pltpu.sync_copy(zer_hbm.at[pl.ds(0, SS)], earows0)
    pltpu.sync_copy(zer_hbm.at[pl.ds(0, SS)], earows1)
    plsc.subcore_barrier()

    pltpu.sync_copy(src3_hbm.at[wid], src2d)
    pltpu.sync_copy(dst3_hbm.at[wid], dst2d)

    nsuper = lax.select(wid == NW - 1, (E - (NW - 1) * EPT) // SS, EPT // SS)
    lane = lax.iota(jnp.int32, 16)
    rhalf = lax.shift_right_logical(lane, 1)
    cpair = lax.bitwise_and(lane, 1)
    col2 = jnp.zeros((16,), jnp.int32) + 2
    ones = jnp.ones((16,), F32)
    NCH = SS // CH

    def issue_gathers(c, xb, ea2d, sg):
        for j in range(NCH):
            q = c * NCH + j
            pltpu.async_copy(xm_hbm.at[src2d.at[q]],
                             xb.at[pl.ds(j * CH, CH)], sg)
        pltpu.async_copy(ea_hbm.at[pl.ds(wid * EPT + c * SS, SS)], ea2d, sg)

    def half(c, xb, eb, ea2d, sg, ss):
        @pl.when(c < nsuper)
        def _():
            for j in range(NCH):
                q = c * NCH + j
                pltpu.make_async_copy(xm_hbm.at[src2d.at[q]],
                                      xb.at[pl.ds(j * CH, CH)], sg).wait()
            pltpu.make_async_copy(ea_hbm.at[pl.ds(wid * EPT + c * SS, SS)],
                                  ea2d, sg).wait()
            for g in range(SS // 8):
                rows = rhalf + g * 8
                pairs = plsc.load_gather(ea2d, [rows, cpair])
                plsc.store_scatter(eb, [rows, cpair], pairs)
            for g in range(SS // 16):
                plsc.store_scatter(eb, [lane + g * 16, col2], ones)
            for j in range(NCH):
                q = c * NCH + j
                pltpu.async_copy(xb.at[pl.ds(j * CH, CH)],
                                 aggsh.at[dst2d.at[q]], ss, add=True)
                pltpu.async_copy(eb.at[pl.ds(j * CH, CH)],
                                 t2sh.at[dst2d.at[q]], ss, add=True)
            for j in range(NCH):
                q = c * NCH + j
                pltpu.make_async_copy(xb.at[pl.ds(j * CH, CH)],
                                      aggsh.at[dst2d.at[q]], ss).wait()
                pltpu.make_async_copy(eb.at[pl.ds(j * CH, CH)],
                                      t2sh.at[dst2d.at[q]], ss).wait()

            @pl.when(c + 2 < nsuper)
            def _():
                issue_gathers(c + 2, xb, ea2d, sg)

    issue_gathers(0, xmbuf0, ea2d0, sg0)
    issue_gathers(1, xmbuf1, ea2d1, sg1)

    def sbody(s2, carry):
        half(2 * s2, xmbuf0, earows0, ea2d0, sg0, ss0)
        half(2 * s2 + 1, xmbuf1, earows1, ea2d1, sg1, ss1)
        return carry

    lax.fori_loop(0, EPT // SS // 2, sbody, 0)
    plsc.subcore_barrier()

    pltpu.sync_copy(aggsh.at[pl.ds(sid * STRIPE, STRIPE)], stripe)
    pltpu.sync_copy(stripe, agg_hbm.at[cid, pl.ds(sid * STRIPE, STRIPE)])
    pltpu.sync_copy(t2sh.at[pl.ds(sid * STRIPE, STRIPE)], stripe)
    pltpu.sync_copy(stripe, t2_hbm.at[cid, pl.ds(sid * STRIPE, STRIPE)])


@functools.cache
def _build_edge_kernel():
    return functools.partial(
        pl.kernel,
        mesh=plsc.VectorSubcoreMesh(core_axis_name="c", subcore_axis_name="s"),
        out_type=[jax.ShapeDtypeStruct((NC, NROW, EMB), F32),
                  jax.ShapeDtypeStruct((NC, NROW, EMB), F32)],
        compiler_params=pltpu.CompilerParams(use_tc_tiling_on_sc=False,
                                             needs_layout_passes=False),
        scratch_types=[
            pltpu.VMEM((EPT // CH, CH), jnp.int32),
            pltpu.VMEM((EPT // CH, CH), jnp.int32),
            pltpu.VMEM((SS, 2), F32),
            pltpu.VMEM((SS, 2), F32),
            pltpu.VMEM((SS, EMB), F32),
            pltpu.VMEM((SS, EMB), F32),
            pltpu.VMEM((SS, EMB), F32),
            pltpu.VMEM((SS, EMB), F32),
            pltpu.VMEM((STRIPE, EMB), F32),
            pltpu.VMEM_SHARED((NROW, EMB), F32),
            pltpu.VMEM_SHARED((NROW, EMB), F32),
            pltpu.SemaphoreType.DMA,
            pltpu.SemaphoreType.DMA,
            pltpu.SemaphoreType.DMA,
            pltpu.SemaphoreType.DMA,
        ],
    )(_edge_body)


def _ns_body(agg8, t28, m, xsb8, ns8_o):
    a = agg8[...]
    t = t28[...]
    mv = m[...]
    ts = t[0, :N // 8, :] + t[1, :N // 8, :]
    base = a[0, :N // 8, :] + a[1, :N // 8, :] + xsb8[...]
    for j in range(8):
        tc = jnp.dot(ts[:, j * EMB:(j + 1) * EMB], mv,
                     preferred_element_type=F32)
        ns8_o[:, j * EMB:(j + 1) * EMB] = jax.nn.relu(
            base[:, j * EMB:(j + 1) * EMB] + tc)


BK = 6400
NB = N * EMB // BK


def _head_body(flat, w2, us, w2u, b2, w3, b3, w4, b4, out, acc):
    i = pl.program_id(0)

    @pl.when(i == 0)
    def _init():
        acc[...] = jnp.zeros_like(acc)

    acc[...] += jnp.dot(flat[...], w2[...], preferred_element_type=F32)

    @pl.when(i == NB - 1)
    def _tail():
        user = jnp.dot(us[...], w2u[...], preferred_element_type=F32)
        h = jax.nn.relu(acc[...] + user + b2[...])
        h = jax.nn.relu(jnp.dot(h, w3[...], preferred_element_type=F32) + b3[...])
        logits = jnp.dot(h, w4[...], preferred_element_type=F32) + b4[...]
        m = jnp.max(logits, axis=1, keepdims=True)
        ex = jnp.exp(logits - m)
        out[...] = ex / jnp.sum(ex, axis=1, keepdims=True)


def _edge_agg(xm, src3, dst3, edge_attr):
    return _build_edge_kernel()(xm, src3, dst3, edge_attr,
                                jnp.zeros((STRIPE, EMB), F32))


def kernel(x, edge_index, edge_attr, user_s,
           Wm1, bm1, We1, be1, Ws1, bs1,
           Wm2, bm2, We2, be2, Ws2, bs2,
           W1, b1, W2, b2, W3, b3, W4, b4):
    X8 = x.reshape(N // 8, 8 * x.shape[1])
    xm8, xsb8, us = pl.pallas_call(
        _prep_body,
        out_shape=[
            jax.ShapeDtypeStruct((N // 8, 128), F32),
            jax.ShapeDtypeStruct((N // 8, 128), F32),
            jax.ShapeDtypeStruct((1, 64), F32),
        ],
    )(X8, Wm2, Ws2, bs2[None], user_s[None], W1, b1[None])

    xm = xm8.reshape(N, EMB)
    src3 = jnp.pad(edge_index[0], (0, PADE - E)).reshape(NW, EPT // CH, CH)
    dst3 = jnp.pad(edge_index[1], (0, PADE - E)).reshape(NW, EPT // CH, CH)
    agg, t2 = _edge_agg(xm, src3, dst3, edge_attr)

    M = jnp.zeros((EMB, EMB), F32)
    M = M.at[0].set(We2[0]).at[1].set(We2[1]).at[2].set(bm2 + be2)
    agg8 = agg.reshape(NC, NROW // 8, 128)
    t28 = t2.reshape(NC, NROW // 8, 128)
    ns8 = pl.pallas_call(
        _ns_body,
        out_shape=jax.ShapeDtypeStruct((N // 8, 128), F32),
    )(agg8, t28, M, xsb8)

    flat = ns8.reshape(1, N * EMB)
    W2u = lax.slice(W2, (N * EMB, 0), (N * EMB + 64, 256))
    out = pl.pallas_call(
        _head_body,
        grid=(NB,),
        in_specs=[
            pl.BlockSpec((1, BK), lambda i: (0, i)),
            pl.BlockSpec((BK, 256), lambda i: (i, 0)),
            pl.BlockSpec((1, 64), lambda i: (0, 0)),
            pl.BlockSpec((64, 256), lambda i: (0, 0)),
            pl.BlockSpec((1, 256), lambda i: (0, 0)),
            pl.BlockSpec((256, 128), lambda i: (0, 0)),
            pl.BlockSpec((1, 128), lambda i: (0, 0)),
            pl.BlockSpec((128, N), lambda i: (0, 0)),
            pl.BlockSpec((1, N), lambda i: (0, 0)),
        ],
        out_specs=pl.BlockSpec((1, N), lambda i: (0, 0)),
        out_shape=jax.ShapeDtypeStruct((1, N), F32),
        scratch_shapes=[pltpu.VMEM((1, 256), F32)],
        compiler_params=pltpu.CompilerParams(
            dimension_semantics=("arbitrary",)),
    )(flat, W2, us, W2u, b2[None], W3, b3[None], W4, b4[None])
    return out


# speedup vs baseline: 2.7142x; 2.7142x over previous
import functools

import jax
import jax.numpy as jnp
from jax import lax
from jax.experimental import pallas as pl
from jax.experimental.pallas import tpu as pltpu
from jax.experimental.pallas import tpu_sc as plsc

F32 = jnp.float32

N = 10000
E = 320000
EMB = 16
NC, NS = 2, 16
NW = NC * NS
EPT = 10240
PADE = NW * EPT
CH = 128
SS = 512
NROW = 10112
STRIPE = NROW // NS


def _prep_body(x8, wm2, ws2, bs2, u, w1, b1, xm8_o, xsb8_o, us_o):
    x8v = x8[...]
    wm = wm2[...]
    ws = ws2[...]
    bs = bs2[...]
    for j in range(8):
        xj = x8v[:, j * 128:(j + 1) * 128]
        xm8_o[:, j * EMB:(j + 1) * EMB] = jnp.dot(
            xj, wm, preferred_element_type=F32)
        xsb8_o[:, j * EMB:(j + 1) * EMB] = jnp.dot(
            xj, ws, preferred_element_type=F32) + bs
    us_o[...] = jax.nn.relu(
        jnp.dot(u[...], w1[...], preferred_element_type=F32) + b1[...])


def _edge_body(xm_hbm, src3_hbm, dst3_hbm, ea0_hbm, ea1_hbm, zer_hbm,
               agg_hbm, t2_hbm,
               src2d, dst2d, ea0buf, ea1buf, earows0, earows1,
               xmbuf0, xmbuf1, stripe, aggsh, t2sh, sg0, sg1, ss0, ss1):
    cid = lax.axis_index("c")
    sid = lax.axis_index("s")
    wid = sid * NC + cid

    pltpu.sync_copy(zer_hbm, stripe)
    pltpu.sync_copy(stripe, aggsh.at[pl.ds(sid * STRIPE, STRIPE)])
    pltpu.sync_copy(stripe, t2sh.at[pl.ds(sid * STRIPE, STRIPE)])
    pltpu.sync_copy(zer_hbm.at[pl.ds(0, SS)], earows0)
    pltpu.sync_copy(zer_hbm.at[pl.ds(0, SS)], earows1)
    plsc.subcore_barrier()

    pltpu.sync_copy(src3_hbm.at[wid], src2d)
    pltpu.sync_copy(dst3_hbm.at[wid], dst2d)
    pltpu.sync_copy(ea0_hbm.at[pl.ds(wid * EPT, EPT)], ea0buf)
    pltpu.sync_copy(ea1_hbm.at[pl.ds(wid * EPT, EPT)], ea1buf)

    nsuper = lax.select(wid == NW - 1, (E - (NW - 1) * EPT) // SS, EPT // SS)
    lane = lax.iota(jnp.int32, 16)
    col0 = jnp.zeros((16,), jnp.int32)
    ones = jnp.ones((16,), F32)
    NCH = SS // CH

    def issue_gathers(c, xb, sg):
        for j in range(NCH):
            q = c * NCH + j
            pltpu.async_copy(xm_hbm.at[src2d.at[q]],
                             xb.at[pl.ds(j * CH, CH)], sg)

    def half(c, xb, eb, sg, ss):
        @pl.when(c < nsuper)
        def _():
            for g in range(SS // 16):
                rows = lane + g * 16
                e0 = ea0buf[pl.ds(c * SS + g * 16, 16)]
                e1 = ea1buf[pl.ds(c * SS + g * 16, 16)]
                plsc.store_scatter(eb, [rows, col0], e0)
                plsc.store_scatter(eb, [rows, col0 + 1], e1)
                plsc.store_scatter(eb, [rows, col0 + 2], ones)
            for j in range(NCH):
                q = c * NCH + j
                pltpu.make_async_copy(xm_hbm.at[src2d.at[q]],
                                      xb.at[pl.ds(j * CH, CH)], sg).wait()
            for j in range(NCH):
                q = c * NCH + j
                pltpu.async_copy(xb.at[pl.ds(j * CH, CH)],
                                 aggsh.at[dst2d.at[q]], ss, add=True)
                pltpu.async_copy(eb.at[pl.ds(j * CH, CH)],
                                 t2sh.at[dst2d.at[q]], ss, add=True)
            for j in range(NCH):
                q = c * NCH + j
                pltpu.make_async_copy(xb.at[pl.ds(j * CH, CH)],
                                      aggsh.at[dst2d.at[q]], ss).wait()
                pltpu.make_async_copy(eb.at[pl.ds(j * CH, CH)],
                                      t2sh.at[dst2d.at[q]], ss).wait()

            @pl.when(c + 2 < nsuper)
            def _():
                issue_gathers(c + 2, xb, sg)

    issue_gathers(0, xmbuf0, sg0)
    issue_gathers(1, xmbuf1, sg1)

    def sbody(s2, carry):
        half(2 * s2, xmbuf0, earows0, sg0, ss0)
        half(2 * s2 + 1, xmbuf1, earows1, sg1, ss1)
        return carry

    lax.fori_loop(0, EPT // SS // 2, sbody, 0)
    plsc.subcore_barrier()

    pltpu.sync_copy(aggsh.at[pl.ds(sid * STRIPE, STRIPE)], stripe)
    pltpu.sync_copy(stripe, agg_hbm.at[cid, pl.ds(sid * STRIPE, STRIPE)])
    pltpu.sync_copy(t2sh.at[pl.ds(sid * STRIPE, STRIPE)], stripe)
    pltpu.sync_copy(stripe, t2_hbm.at[cid, pl.ds(sid * STRIPE, STRIPE)])


@functools.cache
def _build_edge_kernel():
    return functools.partial(
        pl.kernel,
        mesh=plsc.VectorSubcoreMesh(core_axis_name="c", subcore_axis_name="s"),
        out_type=[jax.ShapeDtypeStruct((NC, NROW, EMB), F32),
                  jax.ShapeDtypeStruct((NC, NROW, EMB), F32)],
        compiler_params=pltpu.CompilerParams(use_tc_tiling_on_sc=False,
                                             needs_layout_passes=False),
        scratch_types=[
            pltpu.VMEM((EPT // CH, CH), jnp.int32),
            pltpu.VMEM((EPT // CH, CH), jnp.int32),
            pltpu.VMEM((EPT,), F32),
            pltpu.VMEM((EPT,), F32),
            pltpu.VMEM((SS, EMB), F32),
            pltpu.VMEM((SS, EMB), F32),
            pltpu.VMEM((SS, EMB), F32),
            pltpu.VMEM((SS, EMB), F32),
            pltpu.VMEM((STRIPE, EMB), F32),
            pltpu.VMEM_SHARED((NROW, EMB), F32),
            pltpu.VMEM_SHARED((NROW, EMB), F32),
            pltpu.SemaphoreType.DMA,
            pltpu.SemaphoreType.DMA,
            pltpu.SemaphoreType.DMA,
            pltpu.SemaphoreType.DMA,
        ],
    )(_edge_body)


def _ns_body(agg8, t28, m, xsb8, ns8_o):
    a = agg8[...]
    t = t28[...]
    mv = m[...]
    ts = t[0, :N // 8, :] + t[1, :N // 8, :]
    base = a[0, :N // 8, :] + a[1, :N // 8, :] + xsb8[...]
    for j in range(8):
        tc = jnp.dot(ts[:, j * EMB:(j + 1) * EMB], mv,
                     preferred_element_type=F32)
        ns8_o[:, j * EMB:(j + 1) * EMB] = jax.nn.relu(
            base[:, j * EMB:(j + 1) * EMB] + tc)


BK = 6400
NB = N * EMB // BK


def _head_body(flat, w2, us, w2u, b2, w3, b3, w4, b4, out, acc):
    i = pl.program_id(0)

    @pl.when(i == 0)
    def _init():
        acc[...] = jnp.zeros_like(acc)

    acc[...] += jnp.dot(flat[...], w2[...], preferred_element_type=F32)

    @pl.when(i == NB - 1)
    def _tail():
        user = jnp.dot(us[...], w2u[...], preferred_element_type=F32)
        h = jax.nn.relu(acc[...] + user + b2[...])
        h = jax.nn.relu(jnp.dot(h, w3[...], preferred_element_type=F32) + b3[...])
        logits = jnp.dot(h, w4[...], preferred_element_type=F32) + b4[...]
        m = jnp.max(logits, axis=1, keepdims=True)
        ex = jnp.exp(logits - m)
        out[...] = ex / jnp.sum(ex, axis=1, keepdims=True)


def _edge_agg(xm, src3, dst3, ea0p, ea1p):
    return _build_edge_kernel()(xm, src3, dst3, ea0p, ea1p,
                                jnp.zeros((STRIPE, EMB), F32))


def kernel(x, edge_index, edge_attr, user_s,
           Wm1, bm1, We1, be1, Ws1, bs1,
           Wm2, bm2, We2, be2, Ws2, bs2,
           W1, b1, W2, b2, W3, b3, W4, b4):
    X8 = x.reshape(N // 8, 8 * x.shape[1])
    xm8, xsb8, us = pl.pallas_call(
        _prep_body,
        out_shape=[
            jax.ShapeDtypeStruct((N // 8, 128), F32),
            jax.ShapeDtypeStruct((N // 8, 128), F32),
            jax.ShapeDtypeStruct((1, 64), F32),
        ],
    )(X8, Wm2, Ws2, bs2[None], user_s[None], W1, b1[None])

    xm = xm8.reshape(N, EMB)
    src3 = jnp.pad(edge_index[0], (0, PADE - E)).reshape(NW, EPT // CH, CH)
    dst3 = jnp.pad(edge_index[1], (0, PADE - E)).reshape(NW, EPT // CH, CH)
    ea0p = jnp.pad(edge_attr[:, 0], (0, PADE - E))
    ea1p = jnp.pad(edge_attr[:, 1], (0, PADE - E))
    agg, t2 = _edge_agg(xm, src3, dst3, ea0p, ea1p)

    M = jnp.zeros((EMB, EMB), F32)
    M = M.at[0].set(We2[0]).at[1].set(We2[1]).at[2].set(bm2 + be2)
    agg8 = agg.reshape(NC, NROW // 8, 128)
    t28 = t2.reshape(NC, NROW // 8, 128)
    ns8 = pl.pallas_call(
        _ns_body,
        out_shape=jax.ShapeDtypeStruct((N // 8, 128), F32),
    )(agg8, t28, M, xsb8)

    flat = ns8.reshape(1, N * EMB)
    W2u = lax.slice(W2, (N * EMB, 0), (N * EMB + 64, 256))
    out = pl.pallas_call(
        _head_body,
        grid=(NB,),
        in_specs=[
            pl.BlockSpec((1, BK), lambda i: (0, i)),
            pl.BlockSpec((BK, 256), lambda i: (i, 0)),
            pl.BlockSpec((1, 64), lambda i: (0, 0)),
            pl.BlockSpec((64, 256), lambda i: (0, 0)),
            pl.BlockSpec((1, 256), lambda i: (0, 0)),
            pl.BlockSpec((256, 128), lambda i: (0, 0)),
            pl.BlockSpec((1, 128), lambda i: (0, 0)),
            pl.BlockSpec((128, N), lambda i: (0, 0)),
            pl.BlockSpec((1, N), lambda i: (0, 0)),
        ],
        out_specs=pl.BlockSpec((1, N), lambda i: (0, 0)),
        out_shape=jax.ShapeDtypeStruct((1, N), F32),
        scratch_shapes=[pltpu.VMEM((1, 256), F32)],
        compiler_params=pltpu.CompilerParams(
            dimension_semantics=("arbitrary",)),
    )(flat, W2, us, W2u, b2[None], W3, b3[None], W4, b4[None])
    return out
